# baseline (speedup 1.0000x reference)
import functools

import jax
import jax.numpy as jnp
from jax import lax
from jax.experimental import pallas as pl
from jax.experimental.pallas import tpu as pltpu
from jax.experimental.pallas import tpu_sc as plsc

TRAIN_START = 900000
DIM = 32
LANES = 16

NC = 2
NS = 16
NW = NC * NS

CH = 512
GBLK = 128
NBLK = CH // GBLK


def _body(n_chunks, frozen_hbm, trainable_hbm, idx_hbm, out_hbm,
          idx_v, fidx_v, tidx_v, tbuf, obuf, semf, semt, semo):
    c = lax.axis_index("c")
    s = lax.axis_index("s")
    wid = s * NC + c
    base = wid * (n_chunks * CH)

    def issue(p, ci):
        cbase = base + ci * CH
        poff = p * CH
        pltpu.sync_copy(idx_hbm.at[pl.ds(cbase, CH)], idx_v.at[pl.ds(poff, CH)])
        for g in range(CH // LANES):
            iv = idx_v[pl.ds(poff + g * LANES, LANES)]
            is_t = iv >= TRAIN_START
            dummy = lax.iota(jnp.int32, LANES) + (wid * CH + g * LANES)
            row = p * NBLK + g // 8
            fidx_v[row, pl.ds((g % 8) * LANES, LANES)] = jnp.where(is_t, dummy, iv)
            tidx_v[row, pl.ds((g % 8) * LANES, LANES)] = jnp.where(
                is_t, iv - TRAIN_START, dummy)
        for j in range(NBLK):
            pltpu.async_copy(
                frozen_hbm.at[fidx_v.at[p * NBLK + j]],
                obuf.at[pl.ds(poff + j * GBLK, GBLK)], semf)
            pltpu.async_copy(
                trainable_hbm.at[tidx_v.at[p * NBLK + j]],
                tbuf.at[pl.ds(poff + j * GBLK, GBLK)], semt)

    def wait_gathers(q):
        qoff = q * CH
        pltpu.make_async_copy(
            frozen_hbm.at[pl.ds(0, CH)], obuf.at[pl.ds(qoff, CH)], semf).wait()
        pltpu.make_async_copy(
            trainable_hbm.at[pl.ds(0, CH)], tbuf.at[pl.ds(qoff, CH)], semt).wait()

    def patch_and_store(q, ci):
        qoff = q * CH

        def group_body(g, rc):
            iv = idx_v[pl.ds(qoff + g * LANES, LANES)]
            rbase = qoff + g * LANES
            for k in range(LANES):
                @pl.when(iv[k] >= TRAIN_START)
                def _patch(k=k):
                    r = rbase + k
                    for h in range(DIM // LANES):
                        col = pl.ds(h * LANES, LANES)
                        obuf[r, col] = tbuf[r, col]
            return rc

        lax.fori_loop(0, CH // LANES, group_body, 0)
        pltpu.async_copy(
            obuf.at[pl.ds(qoff, CH)], out_hbm.at[pl.ds(base + ci * CH, CH)], semo)

    def iter_body(ci, carry):
        p = lax.rem(ci, 2)

        @pl.when(ci < n_chunks)
        def _issue():
            @pl.when(ci >= 2)
            def _wait_store():
                pltpu.make_async_copy(
                    obuf.at[pl.ds(p * CH, CH)], out_hbm.at[pl.ds(0, CH)],
                    semo).wait()
            issue(p, ci)

        @pl.when(ci >= 1)
        def _finish():
            q = 1 - p
            wait_gathers(q)
            patch_and_store(q, ci - 1)

        return carry

    lax.fori_loop(0, n_chunks + 1, iter_body, 0)
    pltpu.make_async_copy(
        obuf.at[pl.ds(0, CH)], out_hbm.at[pl.ds(0, CH)], semo).wait()
    pltpu.make_async_copy(
        obuf.at[pl.ds(CH, CH)], out_hbm.at[pl.ds(0, CH)], semo).wait()


def kernel(frozen_weight, trainable_buffer, input_ids):
    b, s = input_ids.shape
    n = b * s
    assert n % (NW * CH) == 0
    n_chunks = n // (NW * CH)
    idx_flat = input_ids.reshape(n)

    k = pl.kernel(
        functools.partial(_body, n_chunks),
        out_type=jax.ShapeDtypeStruct((n, DIM), jnp.float32),
        mesh=plsc.VectorSubcoreMesh(core_axis_name="c", subcore_axis_name="s"),
        compiler_params=pltpu.CompilerParams(use_tc_tiling_on_sc=False),
        scratch_types=[
            pltpu.VMEM((2 * CH,), jnp.int32),
            pltpu.VMEM((2 * NBLK, GBLK), jnp.int32),
            pltpu.VMEM((2 * NBLK, GBLK), jnp.int32),
            pltpu.VMEM((2 * CH, DIM), jnp.float32),
            pltpu.VMEM((2 * CH, DIM), jnp.float32),
            pltpu.SemaphoreType.DMA,
            pltpu.SemaphoreType.DMA,
            pltpu.SemaphoreType.DMA,
        ],
    )
    out = k(frozen_weight, trainable_buffer, idx_flat)
    return out.reshape(b, s, DIM)


# speedup vs baseline: 1.0360x; 1.0360x over previous
import functools

import jax
import jax.numpy as jnp
from jax import lax
from jax.experimental import pallas as pl
from jax.experimental.pallas import tpu as pltpu
from jax.experimental.pallas import tpu_sc as plsc

TRAIN_START = 900000
DIM = 32
LANES = 16

NC = 2
NS = 16
NW = NC * NS

CH = 512
GBLK = 128
NBLK = CH // GBLK
TB = LANES


def _body(n_chunks, frozen_hbm, trainable_hbm, idx_hbm, out_hbm,
          idx_v, fidx_v, tval_v, tbuf, obuf, semf, semt):
    c = lax.axis_index("c")
    s = lax.axis_index("s")
    wid = s * NC + c
    base = wid * (n_chunks * CH)
    lane = lax.iota(jnp.int32, LANES)

    def chunk_body(ci, carry):
        cbase = base + ci * CH
        pltpu.sync_copy(idx_hbm.at[pl.ds(cbase, CH)], idx_v)

        nt = jnp.int32(0)
        for g in range(CH // LANES):
            iv = idx_v[pl.ds(g * LANES, LANES)]
            is_t = iv >= TRAIN_START
            dummy = lane + (wid * CH + g * LANES)
            fidx_v[g // 8, pl.ds((g % 8) * LANES, LANES)] = jnp.where(
                is_t, dummy, iv)
            packed = ((iv - TRAIN_START) << 9) | (lane + g * LANES)
            cnt = lax.cumsum(is_t.astype(jnp.int32), axis=0)
            pos = jnp.where(is_t, (nt - 1) + cnt, CH + LANES + lane)
            plsc.store_scatter(tval_v, [pos], packed)
            nt = nt + cnt[LANES - 1]

        for j in range(NBLK):
            pltpu.async_copy(
                frozen_hbm.at[fidx_v.at[j]],
                obuf.at[pl.ds(j * GBLK, GBLK)], semf)

        last = jnp.full((LANES,), jnp.maximum(nt - 1, 0), jnp.int32)
        plsc.store_scatter(tval_v, [lane + nt], plsc.load_gather(tval_v, [last]))
        ntb = (nt + (TB - 1)) // TB

        def fire(j, fc):
            tid = tval_v[pl.ds(j * TB, TB)] >> 9
            pltpu.async_copy(
                trainable_hbm.at[tid], tbuf.at[pl.ds(j * TB, TB)], semt)
            return fc

        lax.fori_loop(0, ntb, fire, 0)

        pltpu.make_async_copy(
            frozen_hbm.at[pl.ds(0, CH)], obuf, semf).wait()

        def merge(j, mc):
            pltpu.make_async_copy(
                trainable_hbm.at[pl.ds(0, TB)],
                tbuf.at[pl.ds(0, TB)], semt).wait()
            rows = tval_v[pl.ds(j * TB, TB)] & 511
            lids = lane + j * TB
            for col in range(DIM):
                cs = jnp.full((LANES,), col, jnp.int32)
                vals = plsc.load_gather(tbuf, [lids, cs])
                plsc.store_scatter(obuf, [rows, cs], vals)
            return mc

        lax.fori_loop(0, ntb, merge, 0)

        pltpu.sync_copy(obuf, out_hbm.at[pl.ds(cbase, CH)])
        return carry

    lax.fori_loop(0, n_chunks, chunk_body, 0)


def kernel(frozen_weight, trainable_buffer, input_ids):
    b, s = input_ids.shape
    n = b * s
    assert n % (NW * CH) == 0
    n_chunks = n // (NW * CH)
    idx_flat = input_ids.reshape(n)

    k = pl.kernel(
        functools.partial(_body, n_chunks),
        out_type=jax.ShapeDtypeStruct((n, DIM), jnp.float32),
        mesh=plsc.VectorSubcoreMesh(core_axis_name="c", subcore_axis_name="s"),
        compiler_params=pltpu.CompilerParams(
            use_tc_tiling_on_sc=False, needs_layout_passes=False),
        scratch_types=[
            pltpu.VMEM((CH,), jnp.int32),
            pltpu.VMEM((NBLK, GBLK), jnp.int32),
            pltpu.VMEM((CH + 2 * LANES,), jnp.int32),
            pltpu.VMEM((CH + LANES, DIM), jnp.float32),
            pltpu.VMEM((CH, DIM), jnp.float32),
            pltpu.SemaphoreType.DMA,
            pltpu.SemaphoreType.DMA,
        ],
    )
    out = k(frozen_weight, trainable_buffer, idx_flat)
    return out.reshape(b, s, DIM)


# speedup vs baseline: 1.0959x; 1.0578x over previous
import functools

import jax
import jax.numpy as jnp
from jax import lax
from jax.experimental import pallas as pl
from jax.experimental.pallas import tpu as pltpu
from jax.experimental.pallas import tpu_sc as plsc

TRAIN_START = 900000
DIM = 32
LANES = 16

NC = 2
NS = 16
NW = NC * NS

CH = 512
GBLK = 128
NBLK = CH // GBLK
TB = LANES
TSLOT = CH + 2 * LANES
TBH = CH + LANES


def _body(n_chunks, frozen_hbm, trainable_hbm, idx_hbm, out_hbm,
          idx_v, fidx_v, tval_v, tbuf, obuf, semf, semt, semo):
    c = lax.axis_index("c")
    s = lax.axis_index("s")
    wid = s * NC + c
    base = wid * (n_chunks * CH)
    lane = lax.iota(jnp.int32, LANES)

    def stage_issue(ci):
        p = lax.rem(ci, 2)
        pltpu.sync_copy(idx_hbm.at[pl.ds(base + ci * CH, CH)], idx_v)

        nt = jnp.int32(0)
        for g in range(CH // LANES):
            iv = idx_v[pl.ds(g * LANES, LANES)]
            is_t = iv >= TRAIN_START
            dummy = lane + (wid * CH + g * LANES)
            fidx_v[p * NBLK + g // 8, pl.ds((g % 8) * LANES, LANES)] = (
                jnp.where(is_t, dummy, iv))
            packed = ((iv - TRAIN_START) << 9) | (lane + g * LANES)
            cnt = lax.cumsum(is_t.astype(jnp.int32), axis=0)
            pos = p * TSLOT + jnp.where(
                is_t, (nt - 1) + cnt, CH + LANES + lane)
            plsc.store_scatter(tval_v, [pos], packed)
            nt = nt + cnt[LANES - 1]

        for j in range(NBLK):
            pltpu.async_copy(
                frozen_hbm.at[fidx_v.at[p * NBLK + j]],
                obuf.at[pl.ds(p * CH + j * GBLK, GBLK)], semf)

        last = jnp.full((LANES,), p * TSLOT + jnp.maximum(nt - 1, 0), jnp.int32)
        plsc.store_scatter(
            tval_v, [lane + (p * TSLOT + nt)], plsc.load_gather(tval_v, [last]))
        ntb = (nt + (TB - 1)) // TB

        def fire(j, fc):
            tid = tval_v[pl.ds(p * TSLOT + j * TB, TB)] >> 9
            pltpu.async_copy(
                trainable_hbm.at[tid],
                tbuf.at[pl.ds(p * TBH + j * TB, TB)], semt)
            return fc

        lax.fori_loop(0, ntb, fire, 0)
        return ntb

    def stage_finish(ci, ntb):
        q = lax.rem(ci, 2)
        pltpu.make_async_copy(
            frozen_hbm.at[pl.ds(0, CH)], obuf.at[pl.ds(0, CH)], semf).wait()

        def merge(j, mc):
            pltpu.make_async_copy(
                trainable_hbm.at[pl.ds(0, TB)],
                tbuf.at[pl.ds(0, TB)], semt).wait()
            v = tval_v[pl.ds(q * TSLOT + j * TB, TB)]
            rows = (v & (CH - 1)) + q * CH
            lids = lane + (q * TBH + j * TB)
            for col in range(DIM):
                cs = jnp.full((LANES,), col, jnp.int32)
                vals = plsc.load_gather(tbuf, [lids, cs])
                plsc.store_scatter(obuf, [rows, cs], vals)
            return mc

        lax.fori_loop(0, ntb, merge, 0)
        pltpu.async_copy(
            obuf.at[pl.ds(q * CH, CH)],
            out_hbm.at[pl.ds(base + ci * CH, CH)], semo)

    def iter_body(ci, ntb_prev):
        @pl.when(ci >= 2)
        def _wait_store():
            pltpu.make_async_copy(
                obuf.at[pl.ds(0, CH)], out_hbm.at[pl.ds(0, CH)], semo).wait()

        ntb = stage_issue(ci)

        @pl.when(ci >= 1)
        def _finish_prev():
            stage_finish(ci - 1, ntb_prev)

        return ntb

    ntb_last = lax.fori_loop(0, n_chunks, iter_body, jnp.int32(0))
    stage_finish(n_chunks - 1, ntb_last)
    pltpu.make_async_copy(
        obuf.at[pl.ds(0, CH)], out_hbm.at[pl.ds(0, CH)], semo).wait()
    pltpu.make_async_copy(
        obuf.at[pl.ds(0, CH)], out_hbm.at[pl.ds(0, CH)], semo).wait()


def kernel(frozen_weight, trainable_buffer, input_ids):
    b, s = input_ids.shape
    n = b * s
    assert n % (NW * CH) == 0
    n_chunks = n // (NW * CH)
    idx_flat = input_ids.reshape(n)

    k = pl.kernel(
        functools.partial(_body, n_chunks),
        out_type=jax.ShapeDtypeStruct((n, DIM), jnp.float32),
        mesh=plsc.VectorSubcoreMesh(core_axis_name="c", subcore_axis_name="s"),
        compiler_params=pltpu.CompilerParams(
            use_tc_tiling_on_sc=False, needs_layout_passes=False),
        scratch_types=[
            pltpu.VMEM((CH,), jnp.int32),
            pltpu.VMEM((2 * NBLK, GBLK), jnp.int32),
            pltpu.VMEM((2 * TSLOT,), jnp.int32),
            pltpu.VMEM((2 * TBH, DIM), jnp.float32),
            pltpu.VMEM((2 * CH, DIM), jnp.float32),
            pltpu.SemaphoreType.DMA,
            pltpu.SemaphoreType.DMA,
            pltpu.SemaphoreType.DMA,
        ],
    )
    out = k(frozen_weight, trainable_buffer, idx_flat)
    return out.reshape(b, s, DIM)
